# baseline (speedup 1.0000x reference)
import functools

import jax
import jax.numpy as jnp
from jax import lax
from jax.experimental import pallas as pl
from jax.experimental.pallas import tpu as pltpu
from jax.experimental.pallas import tpu_sc as plsc

GROUP = 128
K = 4


def _emb_call(G, D, gpw):
    mesh = plsc.VectorSubcoreMesh(core_axis_name="c", subcore_axis_name="s")

    @functools.partial(
        pl.kernel,
        mesh=mesh,
        out_type=jax.ShapeDtypeStruct((G, GROUP, D), jnp.float32),
        scratch_types=[
            pltpu.VMEM((K, GROUP), jnp.int32),
            pltpu.VMEM((K, GROUP, D), jnp.float32),
            pltpu.SemaphoreType.DMA,
        ],
        compiler_params=pltpu.CompilerParams(use_tc_tiling_on_sc=False),
    )
    def emb(table_hbm, idx_hbm, out_hbm, idx_v, rows_v, sem):
        wid = lax.axis_index("s") * 2 + lax.axis_index("c")
        base = wid * gpw

        def chunk(c, carry):
            g0 = base + c * K
            pltpu.sync_copy(idx_hbm.at[pl.ds(g0, K)], idx_v)
            copies = [
                pltpu.async_copy(table_hbm.at[idx_v.at[j]], rows_v.at[j], sem)
                for j in range(K)
            ]
            for cp in copies:
                cp.wait()
            pltpu.sync_copy(rows_v, out_hbm.at[pl.ds(g0, K)])
            return carry

        lax.fori_loop(0, gpw // K, chunk, 0)

    return emb


def kernel(x, table):
    B, H = x.shape
    V, D = table.shape
    n = B * H
    G = n // GROUP
    NW = 32
    gpw = G // NW
    idx2d = x.reshape(G, GROUP).astype(jnp.int32)
    out = _emb_call(G, D, gpw)(table, idx2d)
    return out.reshape(B, H, D)


# speedup vs baseline: 1.0454x; 1.0454x over previous
import functools

import jax
import jax.numpy as jnp
from jax import lax
from jax.experimental import pallas as pl
from jax.experimental.pallas import tpu as pltpu
from jax.experimental.pallas import tpu_sc as plsc

GROUP = 128
K = 5
NW = 32


def _emb_call(G, D, gpw):
    nch = gpw // K
    mesh = plsc.VectorSubcoreMesh(core_axis_name="c", subcore_axis_name="s")

    @functools.partial(
        pl.kernel,
        mesh=mesh,
        out_type=jax.ShapeDtypeStruct((G, GROUP, D), jnp.float32),
        scratch_types=[
            pltpu.VMEM((gpw, GROUP), jnp.int32),
            pltpu.VMEM((2, K, GROUP, D), jnp.float32),
            pltpu.SemaphoreType.DMA,
            pltpu.SemaphoreType.DMA,
            pltpu.SemaphoreType.DMA,
            pltpu.SemaphoreType.DMA,
        ],
        compiler_params=pltpu.CompilerParams(use_tc_tiling_on_sc=False),
    )
    def emb(table_hbm, idx_hbm, out_hbm, idx_v, rows_v, g0sem, g1sem,
            w0sem, w1sem):
        wid = lax.axis_index("s") * 2 + lax.axis_index("c")
        base = wid * gpw
        gsems = (g0sem, g1sem)
        wsems = (w0sem, w1sem)

        def fire_gathers(c, s):
            for j in range(K):
                pltpu.async_copy(
                    table_hbm.at[idx_v.at[c * K + j]],
                    rows_v.at[s].at[j],
                    gsems[s],
                )

        def drain_gathers(c, s):
            for j in range(K):
                pltpu.make_async_copy(
                    table_hbm.at[idx_v.at[c * K + j]],
                    rows_v.at[s].at[j],
                    gsems[s],
                ).wait()

        def fire_write(c, s):
            pltpu.async_copy(
                rows_v.at[s], out_hbm.at[pl.ds(base + c * K, K)], wsems[s])

        def wait_write(c, s):
            pltpu.make_async_copy(
                rows_v.at[s], out_hbm.at[pl.ds(base + c * K, K)],
                wsems[s]).wait()

        pltpu.sync_copy(idx_hbm.at[pl.ds(base, gpw)], idx_v)
        fire_gathers(0, 0)
        fire_gathers(1, 1)

        def body(i, carry):
            cc = i * 2
            for b in range(2):
                c = cc + b
                drain_gathers(c, b)
                fire_write(c, b)
                wait_write(c, b)
                fire_gathers(c + 2, b)
            return carry

        lax.fori_loop(0, (nch - 2) // 2, body, 0)

        for b in range(2):
            c = nch - 2 + b
            drain_gathers(c, b)
            fire_write(c, b)
        for b in range(2):
            wait_write(nch - 2 + b, b)

    return emb


def kernel(x, table):
    B, H = x.shape
    V, D = table.shape
    G = (B * H) // GROUP
    gpw = G // NW
    idx2d = x.reshape(G, GROUP).astype(jnp.int32)
    out = _emb_call(G, D, gpw)(table, idx2d)
    return out.reshape(B, H, D)
